# baseline (speedup 1.0000x reference)
import functools

import jax
import jax.numpy as jnp
from jax import lax
from jax.experimental import pallas as pl
from jax.experimental.pallas import tpu as pltpu
from jax.experimental.pallas import tpu_sc as plsc

B = 1024
L = 20
N_CAT = 26
N_CONT = 20
VOCAB = 1000
HID = 128
OUT = 128

NT = B * L
NC, NS, LANES = 2, 16, 16
NW = NC * NS
TPW = NT // NW
T_CHUNK = 4
ROWS = N_CAT * T_CHUNK
N_CHUNKS = TPW // T_CHUNK
VREGS = OUT // LANES
NBUF = 2


def _project_kernel(tab_ref, w_ref, p_ref):
    p_ref[...] = jnp.dot(tab_ref[0], w_ref[...],
                         preferred_element_type=jnp.float32)


def _project_tables(tables, w_out):
    return pl.pallas_call(
        _project_kernel,
        grid=(N_CAT,),
        in_specs=[
            pl.BlockSpec((1, VOCAB, HID), lambda i: (i, 0, 0)),
            pl.BlockSpec((HID, OUT), lambda i: (i, 0)),
        ],
        out_specs=pl.BlockSpec((VOCAB, OUT), lambda i: (i, 0)),
        out_shape=jax.ShapeDtypeStruct((N_CAT * VOCAB, OUT), jnp.float32),
    )(tables, w_out)


def _fidx_kernel(x3_ref, fidx_ref):
    x26 = jnp.reshape(x3_ref[...][:, :, :N_CAT], (NT, N_CAT))
    fidx_ref[...] = x26 + VOCAB * lax.broadcasted_iota(
        jnp.int32, (NT, N_CAT), 1)


def _make_fidx(x3):
    return pl.pallas_call(
        _fidx_kernel,
        out_shape=jax.ShapeDtypeStruct((NT, N_CAT), jnp.int32),
    )(x3)


def _base_kernel(x3_ref, gamma_ref, beta_ref, wc_ref, bc_ref,
                 woc_ref, bo_ref, base_ref):
    cont3 = x3_ref[...][:, :, N_CAT:].astype(jnp.float32)
    denom = float(B * N_CONT)
    mean = jnp.sum(jnp.sum(cont3, axis=0), axis=1) / denom
    dev = cont3 - mean[None, :, None]
    var = jnp.sum(jnp.sum(dev * dev, axis=0), axis=1) / denom
    s = gamma_ref[0] / jnp.sqrt(var + 1e-5)
    t = beta_ref[0] - mean * s
    normed = cont3 * s[None, :, None] + t[None, :, None]
    n2 = jnp.reshape(normed, (NT, N_CONT))
    w2 = jnp.dot(wc_ref[...], woc_ref[...],
                 preferred_element_type=jnp.float32)
    c = jnp.dot(bc_ref[...], woc_ref[...],
                preferred_element_type=jnp.float32) + bo_ref[...]
    base_ref[...] = jnp.dot(n2, w2, preferred_element_type=jnp.float32) + c


def _make_base(x3, gamma, beta, w_cont, b_cont, w_out_c, b_out):
    return pl.pallas_call(
        _base_kernel,
        out_shape=jax.ShapeDtypeStruct((NT, OUT), jnp.float32),
    )(x3, gamma, beta, w_cont, b_cont, w_out_c, b_out)


def _tree_sum(vals):
    while len(vals) > 1:
        nxt = [vals[i] + vals[i + 1] for i in range(0, len(vals) - 1, 2)]
        if len(vals) % 2:
            nxt.append(vals[-1])
        vals = nxt
    return vals[0]


N_CHUNK2 = N_CHUNKS // 2


def _gather_sum_body(p_hbm, fidx_hbm, out_hbm, idx_v, rows_v, out_v,
                     sem0, sem1):
    wid = lax.axis_index("s") * NC + lax.axis_index("c")
    tok0 = wid * TPW
    sems = (sem0, sem1)

    pltpu.sync_copy(fidx_hbm.at[pl.ds(wid * N_CHUNKS, N_CHUNKS)], idx_v)

    def start(c, par):
        for k in range(2):
            pltpu.async_copy(p_hbm.at[idx_v.at[2 * c + k]],
                             rows_v.at[par, pl.ds(k * ROWS, ROWS)],
                             sems[par])

    def wait(c, par):
        for k in range(2):
            pltpu.make_async_copy(p_hbm.at[idx_v.at[2 * c + k]],
                                  rows_v.at[par, pl.ds(k * ROWS, ROWS)],
                                  sems[par]).wait()

    start(0, 0)

    @pl.loop(0, N_CHUNK2, step=NBUF)
    def _chunk(ci):
        for par in range(NBUF):
            c = ci + par
            nxt = c + 1

            @pl.when(nxt < N_CHUNK2)
            def _():
                start(nxt, 1 - par)

            wait(c, par)
            for t in range(2 * T_CHUNK):
                for v in range(VREGS):
                    sl = pl.ds(v * LANES, LANES)
                    out_v[t, sl] = _tree_sum(
                        [rows_v[par, t * N_CAT + r, sl]
                         for r in range(N_CAT)])
            pltpu.sync_copy(
                out_v, out_hbm.at[pl.ds(tok0 + c * 2 * T_CHUNK,
                                        2 * T_CHUNK)])


def _gather_sum(p, fidx):
    mesh = plsc.VectorSubcoreMesh(core_axis_name="c", subcore_axis_name="s",
                                  num_cores=NC, num_subcores=NS)
    f = pl.kernel(
        _gather_sum_body,
        out_type=jax.ShapeDtypeStruct((NT, OUT), jnp.float32),
        mesh=mesh,
        scratch_types=[
            pltpu.VMEM((N_CHUNKS, ROWS), jnp.int32),
            pltpu.VMEM((NBUF, 2 * ROWS, OUT), jnp.float32),
            pltpu.VMEM((2 * T_CHUNK, OUT), jnp.float32),
            pltpu.SemaphoreType.DMA,
            pltpu.SemaphoreType.DMA,
        ],
    )
    return f(p, fidx)


def _combine_kernel(cat_ref, b_ref, o_ref):
    o_ref[...] = cat_ref[...] + b_ref[...]


def _combine(cat, base):
    blk = NT // 8
    return pl.pallas_call(
        _combine_kernel,
        grid=(8,),
        in_specs=[
            pl.BlockSpec((blk, OUT), lambda i: (i, 0)),
            pl.BlockSpec((blk, OUT), lambda i: (i, 0)),
        ],
        out_specs=pl.BlockSpec((blk, OUT), lambda i: (i, 0)),
        out_shape=jax.ShapeDtypeStruct((NT, OUT), jnp.float32),
    )(cat, base)


def kernel(input_features, tables, bn_gamma, bn_beta, W_cont, b_cont,
           W_out, b_out):
    p = _project_tables(tables, W_out[:N_CAT * HID])
    fidx = _make_fidx(input_features)
    cat = _gather_sum(p, fidx.reshape(NW * N_CHUNKS, ROWS))
    base = _make_base(
        input_features,
        bn_gamma.reshape(1, N_CONT),
        bn_beta.reshape(1, N_CONT),
        W_cont,
        b_cont.reshape(1, HID),
        W_out[N_CAT * HID:],
        b_out.reshape(1, OUT),
    )
    out = _combine(cat, base)
    return out.reshape(B, L, OUT)


# speedup vs baseline: 1.3226x; 1.3226x over previous
import functools

import jax
import jax.numpy as jnp
from jax import lax
from jax.experimental import pallas as pl
from jax.experimental.pallas import tpu as pltpu
from jax.experimental.pallas import tpu_sc as plsc

B = 1024
L = 20
N_CAT = 26
N_CONT = 20
VOCAB = 1000
HID = 128
OUT = 128

NT = B * L
NC, NS, LANES = 2, 16, 16
NW = NC * NS
TPW = NT // NW
T_CHUNK = 4
ROWS = N_CAT * T_CHUNK
N_CHUNKS = TPW // T_CHUNK
VREGS = OUT // LANES
NBUF = 2


def _project_kernel(tab_ref, w_ref, p_ref):
    for i in range(N_CAT):
        p_ref[pl.ds(i * VOCAB, VOCAB), :] = jnp.dot(
            tab_ref[i], w_ref[pl.ds(i * HID, HID), :],
            preferred_element_type=jnp.float32)


def _project_tables(tables, w_out):
    return pl.pallas_call(
        _project_kernel,
        out_shape=jax.ShapeDtypeStruct((N_CAT * VOCAB, OUT), jnp.float32),
    )(tables, w_out)


def _fidx_kernel(x3_ref, fidx_ref):
    x26 = jnp.reshape(x3_ref[...][:, :, :N_CAT], (NT, N_CAT))
    fidx_ref[...] = x26 + VOCAB * lax.broadcasted_iota(
        jnp.int32, (NT, N_CAT), 1)


def _make_fidx(x3):
    return pl.pallas_call(
        _fidx_kernel,
        out_shape=jax.ShapeDtypeStruct((NT, N_CAT), jnp.int32),
    )(x3)


def _base_kernel(x3_ref, gamma_ref, beta_ref, wc_ref, bc_ref,
                 woc_ref, bo_ref, base_ref):
    cont3 = x3_ref[...][:, :, N_CAT:].astype(jnp.float32)
    denom = float(B * N_CONT)
    mean = jnp.sum(jnp.sum(cont3, axis=0), axis=1) / denom
    dev = cont3 - mean[None, :, None]
    var = jnp.sum(jnp.sum(dev * dev, axis=0), axis=1) / denom
    s = gamma_ref[0] / jnp.sqrt(var + 1e-5)
    t = beta_ref[0] - mean * s
    normed = cont3 * s[None, :, None] + t[None, :, None]
    n2 = jnp.reshape(normed, (NT, N_CONT))
    w2 = jnp.dot(wc_ref[...], woc_ref[...],
                 preferred_element_type=jnp.float32)
    c = jnp.dot(bc_ref[...], woc_ref[...],
                preferred_element_type=jnp.float32) + bo_ref[...]
    base_ref[...] = jnp.dot(n2, w2, preferred_element_type=jnp.float32) + c


def _make_base(x3, gamma, beta, w_cont, b_cont, w_out_c, b_out):
    return pl.pallas_call(
        _base_kernel,
        out_shape=jax.ShapeDtypeStruct((NT, OUT), jnp.float32),
    )(x3, gamma, beta, w_cont, b_cont, w_out_c, b_out)


def _tree_sum(vals):
    while len(vals) > 1:
        nxt = [vals[i] + vals[i + 1] for i in range(0, len(vals) - 1, 2)]
        if len(vals) % 2:
            nxt.append(vals[-1])
        vals = nxt
    return vals[0]


def _gather_sum_body(p_hbm, fidx_hbm, out_hbm, idx_v, rows_v, acc_v,
                     sem0, sem1):
    wid = lax.axis_index("s") * NC + lax.axis_index("c")
    tok0 = wid * TPW
    sems = (sem0, sem1)

    pltpu.sync_copy(fidx_hbm.at[pl.ds(wid * N_CHUNKS, N_CHUNKS)], idx_v)

    def start(c, par):
        pltpu.async_copy(p_hbm.at[idx_v.at[c]], rows_v.at[par], sems[par])

    def wait(c, par):
        pltpu.make_async_copy(p_hbm.at[idx_v.at[c]], rows_v.at[par],
                              sems[par]).wait()

    start(0, 0)

    @pl.loop(0, N_CHUNKS, step=NBUF)
    def _chunk(ci):
        for par in range(NBUF):
            c = ci + par
            nxt = c + 1

            @pl.when(nxt < N_CHUNKS)
            def _():
                start(nxt, 1 - par)

            wait(c, par)
            for t in range(T_CHUNK):
                row = c * T_CHUNK + t
                for v in range(VREGS):
                    sl = pl.ds(v * LANES, LANES)
                    acc_v[row, sl] = _tree_sum(
                        [rows_v[par, t * N_CAT + r, sl]
                         for r in range(N_CAT)])

    pltpu.sync_copy(acc_v, out_hbm.at[pl.ds(tok0, TPW)])


def _gather_sum(p, fidx):
    mesh = plsc.VectorSubcoreMesh(core_axis_name="c", subcore_axis_name="s",
                                  num_cores=NC, num_subcores=NS)
    f = pl.kernel(
        _gather_sum_body,
        out_type=jax.ShapeDtypeStruct((NT, OUT), jnp.float32),
        mesh=mesh,
        scratch_types=[
            pltpu.VMEM((N_CHUNKS, ROWS), jnp.int32),
            pltpu.VMEM((NBUF, ROWS, OUT), jnp.float32),
            pltpu.VMEM((TPW, OUT), jnp.float32),
            pltpu.SemaphoreType.DMA,
            pltpu.SemaphoreType.DMA,
        ],
    )
    return f(p, fidx)


def _combine_kernel(cat_ref, b_ref, o_ref):
    o_ref[...] = cat_ref[...] + b_ref[...]


def _combine(cat, base):
    blk = NT // 8
    return pl.pallas_call(
        _combine_kernel,
        grid=(8,),
        in_specs=[
            pl.BlockSpec((blk, OUT), lambda i: (i, 0)),
            pl.BlockSpec((blk, OUT), lambda i: (i, 0)),
        ],
        out_specs=pl.BlockSpec((blk, OUT), lambda i: (i, 0)),
        out_shape=jax.ShapeDtypeStruct((NT, OUT), jnp.float32),
    )(cat, base)


def kernel(input_features, tables, bn_gamma, bn_beta, W_cont, b_cont,
           W_out, b_out):
    p = _project_tables(tables, W_out[:N_CAT * HID])
    fidx = _make_fidx(input_features)
    cat = _gather_sum(p, fidx.reshape(NW * N_CHUNKS, ROWS))
    base = _make_base(
        input_features,
        bn_gamma.reshape(1, N_CONT),
        bn_beta.reshape(1, N_CONT),
        W_cont,
        b_cont.reshape(1, HID),
        W_out[N_CAT * HID:],
        b_out.reshape(1, OUT),
    )
    out = _combine(cat, base)
    return out.reshape(B, L, OUT)
